# Initial kernel scaffold; baseline (speedup 1.0000x reference)
import functools

import jax
import jax.numpy as jnp
from jax import lax
from jax.experimental import pallas as pl
from jax.experimental.pallas import tpu as pltpu
from jax.experimental.pallas import tpu_sc as plsc

N = 10000
E = 320000
F = 128
H = 32
EMB = 16

NC = 2
NS = 16
NW = NC * NS
B = 128
G = 80
E_PAD = NW * G * B
NBUF = 4

N_ACC = 10016
ZCHUNK = N_ACC // NS
OCHUNK = N // NS

_mesh = plsc.VectorSubcoreMesh(core_axis_name="c", subcore_axis_name="s")


def _deg_body(col_hbm, ones_hbm, zeros_hbm, out_hbm, colv, onesv, shared):
    c = lax.axis_index("c")
    s = lax.axis_index("s")
    wid = c * NS + s
    pltpu.sync_copy(zeros_hbm.at[pl.ds(s * ZCHUNK, ZCHUNK)],
                    shared.at[pl.ds(s * ZCHUNK, ZCHUNK)])
    pltpu.sync_copy(ones_hbm, onesv)
    pltpu.sync_copy(col_hbm.at[pl.ds(wid * G, G)], colv)
    plsc.subcore_barrier()

    @pl.loop(0, G)
    def _(g):
        pltpu.sync_copy(onesv, shared.at[colv.at[g]], add=True)

    plsc.subcore_barrier()
    pltpu.sync_copy(shared.at[pl.ds(s * ZCHUNK, ZCHUNK)],
                    out_hbm.at[c, pl.ds(s * ZCHUNK, ZCHUNK)])


_deg_kernel = functools.partial(
    pl.kernel,
    out_type=jax.ShapeDtypeStruct((NC, N_ACC, 16), jnp.float32),
    mesh=_mesh,
    scratch_types=[
        pltpu.VMEM((G, B), jnp.int32),
        pltpu.VMEM((B, 16), jnp.float32),
        pltpu.VMEM_SHARED((N_ACC, 16), jnp.float32),
    ],
)(_deg_body)


def _agg_body(row_hbm, col_hbm, hs_hbm, zeros_hbm, out_hbm,
              rowv, colv, bufs, shared, sems):
    c = lax.axis_index("c")
    s = lax.axis_index("s")
    wid = c * NS + s
    pltpu.sync_copy(zeros_hbm.at[pl.ds(s * ZCHUNK, ZCHUNK)],
                    shared.at[pl.ds(s * ZCHUNK, ZCHUNK)])
    pltpu.sync_copy(row_hbm.at[pl.ds(wid * G, G)], rowv)
    pltpu.sync_copy(col_hbm.at[pl.ds(wid * G, G)], colv)
    plsc.subcore_barrier()

    for b in range(NBUF):
        pltpu.async_copy(hs_hbm.at[rowv.at[b]], bufs.at[b], sems.at[b])

    @pl.loop(0, G, step=NBUF)
    def _(g0):
        for b in range(NBUF):
            g = g0 + b
            pltpu.make_async_copy(hs_hbm.at[rowv.at[g]], bufs.at[b],
                                  sems.at[b]).wait()
            pltpu.sync_copy(bufs.at[b], shared.at[colv.at[g]], add=True)

            @pl.when(g + NBUF < G)
            def _issue():
                pltpu.async_copy(hs_hbm.at[rowv.at[g + NBUF]], bufs.at[b],
                                 sems.at[b])

    plsc.subcore_barrier()
    pltpu.sync_copy(shared.at[pl.ds(s * OCHUNK, OCHUNK)],
                    out_hbm.at[c, pl.ds(s * OCHUNK, OCHUNK)])


def _make_agg(D):
    return functools.partial(
        pl.kernel,
        out_type=jax.ShapeDtypeStruct((NC, N, D), jnp.float32),
        mesh=_mesh,
        scratch_types=[
            pltpu.VMEM((G, B), jnp.int32),
            pltpu.VMEM((G, B), jnp.int32),
            pltpu.VMEM((NBUF, B, D), jnp.float32),
            pltpu.VMEM_SHARED((N_ACC, D), jnp.float32),
            pltpu.SemaphoreType.DMA((NBUF,)),
        ],
    )(_agg_body)


_agg32 = _make_agg(H)
_agg16 = _make_agg(EMB)

_RB = 1000


def _tc_a_body(x_ref, w_ref, da_ref, db_ref, hs_ref, dinv_ref):
    deg = da_ref[...] + db_ref[...] + 1.0
    dinv = lax.rsqrt(deg)
    h = jnp.dot(x_ref[...], w_ref[...], preferred_element_type=jnp.float32)
    hs_ref[...] = h * dinv
    dinv_ref[...] = dinv


def _tc_a(x, w1, dega, degb):
    return pl.pallas_call(
        _tc_a_body,
        grid=(N // _RB,),
        in_specs=[
            pl.BlockSpec((_RB, F), lambda i: (i, 0)),
            pl.BlockSpec((F, H), lambda i: (0, 0)),
            pl.BlockSpec((_RB, 1), lambda i: (i, 0)),
            pl.BlockSpec((_RB, 1), lambda i: (i, 0)),
        ],
        out_specs=[
            pl.BlockSpec((_RB, H), lambda i: (i, 0)),
            pl.BlockSpec((_RB, 1), lambda i: (i, 0)),
        ],
        out_shape=[
            jax.ShapeDtypeStruct((N, H), jnp.float32),
            jax.ShapeDtypeStruct((N, 1), jnp.float32),
        ],
    )(x, w1, dega, degb)


def _tc_b_body(aa_ref, ab_ref, hs_ref, dinv_ref, b1_ref, w2_ref, out_ref):
    dinv = dinv_ref[...]
    pre = dinv * (aa_ref[...] + ab_ref[...] + hs_ref[...]) + b1_ref[...]
    r = jnp.maximum(pre, 0.0)
    h2 = jnp.dot(r, w2_ref[...], preferred_element_type=jnp.float32)
    out_ref[...] = h2 * dinv


def _tc_b(agg_a, agg_b, hs1, dinv, b1, w2):
    return pl.pallas_call(
        _tc_b_body,
        grid=(N // _RB,),
        in_specs=[
            pl.BlockSpec((_RB, H), lambda i: (i, 0)),
            pl.BlockSpec((_RB, H), lambda i: (i, 0)),
            pl.BlockSpec((_RB, H), lambda i: (i, 0)),
            pl.BlockSpec((_RB, 1), lambda i: (i, 0)),
            pl.BlockSpec((1, H), lambda i: (0, 0)),
            pl.BlockSpec((H, EMB), lambda i: (0, 0)),
        ],
        out_specs=pl.BlockSpec((_RB, EMB), lambda i: (i, 0)),
        out_shape=jax.ShapeDtypeStruct((N, EMB), jnp.float32),
    )(agg_a, agg_b, hs1, dinv, b1, w2)


def _tc_c_body(aa_ref, ab_ref, hs_ref, dinv_ref, b2_ref, out_ref):
    out_ref[...] = (dinv_ref[...] *
                    (aa_ref[...] + ab_ref[...] + hs_ref[...]) + b2_ref[...])


def _tc_c(agg_a, agg_b, hs2, dinv, b2):
    return pl.pallas_call(
        _tc_c_body,
        grid=(N // _RB,),
        in_specs=[
            pl.BlockSpec((_RB, EMB), lambda i: (i, 0)),
            pl.BlockSpec((_RB, EMB), lambda i: (i, 0)),
            pl.BlockSpec((_RB, EMB), lambda i: (i, 0)),
            pl.BlockSpec((_RB, 1), lambda i: (i, 0)),
            pl.BlockSpec((1, EMB), lambda i: (0, 0)),
        ],
        out_specs=pl.BlockSpec((_RB, EMB), lambda i: (i, 0)),
        out_shape=jax.ShapeDtypeStruct((N, EMB), jnp.float32),
    )(agg_a, agg_b, hs2, dinv, b2)


def kernel(x, ei, W1, b1, W2, b2):
    ei = ei.astype(jnp.int32)
    pad = E_PAD - E
    row_r = jnp.concatenate(
        [ei[0], jnp.zeros((pad,), jnp.int32)]).reshape(NW * G, B)
    col_r = jnp.concatenate(
        [ei[1], jnp.full((pad,), N, jnp.int32)]).reshape(NW * G, B)

    ones16 = jnp.ones((B, 16), jnp.float32)
    zeros16 = jnp.zeros((N_ACC, 16), jnp.float32)
    zeros32 = jnp.zeros((N_ACC, H), jnp.float32)

    deg2d = _deg_kernel(col_r, ones16, zeros16)
    dega = deg2d[0, :N, 0:1]
    degb = deg2d[1, :N, 0:1]

    hs1, dinv = _tc_a(x, W1, dega, degb)

    agg1 = _agg32(row_r, col_r, hs1, zeros32)
    hs2 = _tc_b(agg1[0], agg1[1], hs1, dinv, b1.reshape(1, H), W2)

    agg2 = _agg16(row_r, col_r, hs2, zeros16)
    out = _tc_c(agg2[0], agg2[1], hs2, dinv, b2.reshape(1, EMB))
    return out


# speedup vs baseline: 28.4149x; 28.4149x over previous
import functools

import jax
import jax.numpy as jnp
from jax import lax
from jax.experimental import pallas as pl
from jax.experimental.pallas import tpu as pltpu
from jax.experimental.pallas import tpu_sc as plsc

N = 10000
E = 320000
F = 128
H = 32
EMB = 16

NC = 2
NS = 16
NW = NC * NS
B = 128
G = 80
E_PAD = NW * G * B
NBUF = 4

N_ACC = 10112
ZCHUNK = N_ACC // NS

_mesh = plsc.VectorSubcoreMesh(core_axis_name="c", subcore_axis_name="s")
_sc_params = pltpu.CompilerParams(use_tc_tiling_on_sc=False)


def _deg_body(col_hbm, ones_hbm, zeros_hbm, out_hbm, colv, onesv, shared):
    c = lax.axis_index("c")
    s = lax.axis_index("s")
    wid = c * NS + s
    pltpu.sync_copy(zeros_hbm.at[pl.ds(s * ZCHUNK, ZCHUNK)],
                    shared.at[pl.ds(s * ZCHUNK, ZCHUNK)])
    pltpu.sync_copy(ones_hbm, onesv)
    pltpu.sync_copy(col_hbm.at[pl.ds(wid * G, G)], colv)
    plsc.subcore_barrier()

    @pl.loop(0, G)
    def _(g):
        pltpu.sync_copy(onesv, shared.at[colv.at[g]], add=True)

    plsc.subcore_barrier()
    pltpu.sync_copy(shared.at[pl.ds(s * ZCHUNK, ZCHUNK)],
                    out_hbm.at[c, pl.ds(s * ZCHUNK, ZCHUNK)])


_deg_kernel = functools.partial(
    pl.kernel,
    out_type=jax.ShapeDtypeStruct((NC, N_ACC, 16), jnp.float32),
    mesh=_mesh,
    compiler_params=_sc_params,
    scratch_types=[
        pltpu.VMEM((G, B), jnp.int32),
        pltpu.VMEM((B, 16), jnp.float32),
        pltpu.VMEM_SHARED((N_ACC, 16), jnp.float32),
    ],
)(_deg_body)


def _agg_body(row_hbm, col_hbm, hs_hbm, zeros_hbm, out_hbm,
              rowv, colv, bufs, shared, sems):
    c = lax.axis_index("c")
    s = lax.axis_index("s")
    wid = c * NS + s
    pltpu.sync_copy(zeros_hbm.at[pl.ds(s * ZCHUNK, ZCHUNK)],
                    shared.at[pl.ds(s * ZCHUNK, ZCHUNK)])
    pltpu.sync_copy(row_hbm.at[pl.ds(wid * G, G)], rowv)
    pltpu.sync_copy(col_hbm.at[pl.ds(wid * G, G)], colv)
    plsc.subcore_barrier()

    for b in range(NBUF):
        pltpu.async_copy(hs_hbm.at[rowv.at[b]], bufs.at[b], sems.at[b])

    @pl.loop(0, G, step=NBUF)
    def _(g0):
        for b in range(NBUF):
            g = g0 + b
            pltpu.make_async_copy(hs_hbm.at[rowv.at[g]], bufs.at[b],
                                  sems.at[b]).wait()
            pltpu.sync_copy(bufs.at[b], shared.at[colv.at[g]], add=True)

            @pl.when(g + NBUF < G)
            def _issue():
                pltpu.async_copy(hs_hbm.at[rowv.at[g + NBUF]], bufs.at[b],
                                 sems.at[b])

    plsc.subcore_barrier()
    pltpu.sync_copy(shared.at[pl.ds(s * ZCHUNK, ZCHUNK)],
                    out_hbm.at[c, pl.ds(s * ZCHUNK, ZCHUNK)])


def _make_agg(D):
    return functools.partial(
        pl.kernel,
        out_type=jax.ShapeDtypeStruct((NC, N_ACC, D), jnp.float32),
        mesh=_mesh,
        compiler_params=_sc_params,
        scratch_types=[
            pltpu.VMEM((G, B), jnp.int32),
            pltpu.VMEM((G, B), jnp.int32),
            pltpu.VMEM((NBUF, B, D), jnp.float32),
            pltpu.VMEM_SHARED((N_ACC, D), jnp.float32),
            pltpu.SemaphoreType.DMA((NBUF,)),
        ],
    )(_agg_body)


_agg32 = _make_agg(H)
_agg16 = _make_agg(EMB)

_RB = 1000


def _tc_a_body(x_ref, w_ref, da_ref, db_ref, hs_ref, dinv_ref):
    deg = da_ref[...] + db_ref[...] + 1.0
    dinv = lax.rsqrt(deg)
    h = jnp.dot(x_ref[...], w_ref[...], preferred_element_type=jnp.float32)
    hs_ref[...] = h * dinv
    dinv_ref[...] = dinv


def _tc_a(x, w1, dega, degb):
    return pl.pallas_call(
        _tc_a_body,
        grid=(N // _RB,),
        in_specs=[
            pl.BlockSpec((_RB, F), lambda i: (i, 0)),
            pl.BlockSpec((F, H), lambda i: (0, 0)),
            pl.BlockSpec((_RB, 1), lambda i: (i, 0)),
            pl.BlockSpec((_RB, 1), lambda i: (i, 0)),
        ],
        out_specs=[
            pl.BlockSpec((_RB, H), lambda i: (i, 0)),
            pl.BlockSpec((_RB, 1), lambda i: (i, 0)),
        ],
        out_shape=[
            jax.ShapeDtypeStruct((N, H), jnp.float32),
            jax.ShapeDtypeStruct((N, 1), jnp.float32),
        ],
    )(x, w1, dega, degb)


def _tc_b_body(aa_ref, ab_ref, hs_ref, dinv_ref, b1_ref, w2_ref, out_ref):
    dinv = dinv_ref[...]
    pre = dinv * (aa_ref[...] + ab_ref[...] + hs_ref[...]) + b1_ref[...]
    r = jnp.maximum(pre, 0.0)
    h2 = jnp.dot(r, w2_ref[...], preferred_element_type=jnp.float32)
    out_ref[...] = h2 * dinv


def _tc_b(agg_a, agg_b, hs1, dinv, b1, w2):
    return pl.pallas_call(
        _tc_b_body,
        grid=(N // _RB,),
        in_specs=[
            pl.BlockSpec((_RB, H), lambda i: (i, 0)),
            pl.BlockSpec((_RB, H), lambda i: (i, 0)),
            pl.BlockSpec((_RB, H), lambda i: (i, 0)),
            pl.BlockSpec((_RB, 1), lambda i: (i, 0)),
            pl.BlockSpec((1, H), lambda i: (0, 0)),
            pl.BlockSpec((H, EMB), lambda i: (0, 0)),
        ],
        out_specs=pl.BlockSpec((_RB, EMB), lambda i: (i, 0)),
        out_shape=jax.ShapeDtypeStruct((N, EMB), jnp.float32),
    )(agg_a, agg_b, hs1, dinv, b1, w2)


def _tc_c_body(aa_ref, ab_ref, hs_ref, dinv_ref, b2_ref, out_ref):
    out_ref[...] = (dinv_ref[...] *
                    (aa_ref[...] + ab_ref[...] + hs_ref[...]) + b2_ref[...])


def _tc_c(agg_a, agg_b, hs2, dinv, b2):
    return pl.pallas_call(
        _tc_c_body,
        grid=(N // _RB,),
        in_specs=[
            pl.BlockSpec((_RB, EMB), lambda i: (i, 0)),
            pl.BlockSpec((_RB, EMB), lambda i: (i, 0)),
            pl.BlockSpec((_RB, EMB), lambda i: (i, 0)),
            pl.BlockSpec((_RB, 1), lambda i: (i, 0)),
            pl.BlockSpec((1, EMB), lambda i: (0, 0)),
        ],
        out_specs=pl.BlockSpec((_RB, EMB), lambda i: (i, 0)),
        out_shape=jax.ShapeDtypeStruct((N, EMB), jnp.float32),
    )(agg_a, agg_b, hs2, dinv, b2)


def kernel(x, ei, W1, b1, W2, b2):
    ei = ei.astype(jnp.int32)
    pad = E_PAD - E
    row_r = jnp.concatenate(
        [ei[0], jnp.zeros((pad,), jnp.int32)]).reshape(NW * G, B)
    col_r = jnp.concatenate(
        [ei[1], jnp.full((pad,), N, jnp.int32)]).reshape(NW * G, B)

    ones16 = jnp.ones((B, 16), jnp.float32)
    zeros16 = jnp.zeros((N_ACC, 16), jnp.float32)
    zeros32 = jnp.zeros((N_ACC, H), jnp.float32)

    deg2d = _deg_kernel(col_r, ones16, zeros16)
    dega = deg2d[0, :N, 0:1]
    degb = deg2d[1, :N, 0:1]

    hs1, dinv = _tc_a(x, W1, dega, degb)

    agg1 = _agg32(row_r, col_r, hs1, zeros32)
    hs2 = _tc_b(agg1[0, :N], agg1[1, :N], hs1, dinv, b1.reshape(1, H), W2)

    agg2 = _agg16(row_r, col_r, hs2, zeros16)
    out = _tc_c(agg2[0, :N], agg2[1, :N], hs2, dinv, b2.reshape(1, EMB))
    return out
